# baseline (speedup 1.0000x reference)
import functools

import jax
import jax.numpy as jnp
from jax import lax
from jax.experimental import pallas as pl
from jax.experimental.pallas import tpu as pltpu
from jax.experimental.pallas import tpu_sc as plsc

NC = 2
NS = 16
NW = NC * NS
L = 16
KNN = 32

f32 = jnp.float32
i32 = jnp.int32


def _mesh():
    return plsc.VectorSubcoreMesh(core_axis_name="c", subcore_axis_name="s")


def _wid():
    return lax.axis_index("s") * NC + lax.axis_index("c")


def _make_weights(ns_pad, nq_pad, sigma, with_feat):
    chunk = nq_pad // NW
    G = chunk // L
    neg_inv = -1.0 / (2.0 * sigma * sigma)
    K = KNN

    out_type = [jax.ShapeDtypeStruct((nq_pad * K,), f32)]
    if with_feat:
        out_type.append(jax.ShapeDtypeStruct((nq_pad,), f32))
    scratch = [
        pltpu.VMEM((ns_pad,), f32),
        pltpu.VMEM((ns_pad,), f32),
        pltpu.VMEM((ns_pad,), f32),
        pltpu.VMEM((chunk,), f32),
        pltpu.VMEM((chunk,), f32),
        pltpu.VMEM((chunk,), f32),
        pltpu.VMEM((chunk * K,), i32),
        pltpu.VMEM((K, chunk), f32),
        pltpu.VMEM((chunk * K,), f32),
    ]
    if with_feat:
        scratch += [
            pltpu.VMEM((ns_pad,), f32),
            pltpu.VMEM((chunk,), f32),
            pltpu.VMEM((L,), f32),
        ]

    def body(*refs):
        if with_feat:
            (spx, spy, spz, qpx, qpy, qpz, nbf, feat, lift, wout, aout,
             sx_v, sy_v, sz_v, qx_v, qy_v, qz_v, nb_v, w_v, wr_v,
             f_v, a_v, l_v) = refs
        else:
            (spx, spy, spz, qpx, qpy, qpz, nbf, wout,
             sx_v, sy_v, sz_v, qx_v, qy_v, qz_v, nb_v, w_v, wr_v) = refs
        wid = _wid()
        base = wid * chunk
        pltpu.sync_copy(spx, sx_v)
        pltpu.sync_copy(spy, sy_v)
        pltpu.sync_copy(spz, sz_v)
        pltpu.sync_copy(qpx.at[pl.ds(base, chunk)], qx_v)
        pltpu.sync_copy(qpy.at[pl.ds(base, chunk)], qy_v)
        pltpu.sync_copy(qpz.at[pl.ds(base, chunk)], qz_v)
        pltpu.sync_copy(nbf.at[pl.ds(base * K, chunk * K)], nb_v)
        if with_feat:
            pltpu.sync_copy(feat, f_v)
            pltpu.sync_copy(lift, l_v)

        lanes = jnp.arange(L, dtype=i32)

        def group(g, carry):
            off = g * L
            qx = qx_v[pl.ds(off, L)]
            qy = qy_v[pl.ds(off, L)]
            qz = qz_v[pl.ds(off, L)]
            pos0 = (lanes + off) * K
            acc = jnp.zeros((L,), f32)
            facc = jnp.zeros((L,), f32)
            for k in range(K):
                idx = plsc.load_gather(nb_v, [pos0 + k])
                sx = plsc.load_gather(sx_v, [idx])
                sy = plsc.load_gather(sy_v, [idx])
                sz = plsc.load_gather(sz_v, [idx])
                dx = sx - qx
                dy = sy - qy
                dz = sz - qz
                w = jnp.exp((dx * dx + dy * dy + dz * dz) * neg_inv)
                w_v[k, pl.ds(off, L)] = w
                acc = acc + w
                if with_feat:
                    facc = facc + w * plsc.load_gather(f_v, [idx])
            inv = 1.0 / (acc + 1e-8)
            for k in range(K):
                plsc.store_scatter(wr_v, [pos0 + k],
                                   w_v[k, pl.ds(off, L)] * inv)
            if with_feat:
                lv = l_v[pl.ds(0, L)]
                a_v[pl.ds(off, L)] = (lv[0] * facc + lv[1] * acc) * inv
            return carry

        lax.fori_loop(0, G, group, 0)
        pltpu.sync_copy(wr_v, wout.at[pl.ds(base * K, chunk * K)])
        if with_feat:
            pltpu.sync_copy(a_v, aout.at[pl.ds(base, chunk)])

    return pl.kernel(body, out_type=out_type, mesh=_mesh(),
                     scratch_types=scratch,
                     compiler_params=pltpu.CompilerParams(
                         needs_layout_passes=False))


def _sc_weights(spT, qpT, nbf, sigma, feat=None, lift=None):
    ns_pad = spT.shape[1]
    nq_pad = nbf.shape[0] // KNN
    k = _make_weights(ns_pad, nq_pad, sigma, feat is not None)
    args = (spT[0], spT[1], spT[2], qpT[0], qpT[1], qpT[2], nbf)
    if feat is None:
        (wf,) = k(*args)
        return wf.reshape(nq_pad, KNN)
    wf, agg0 = k(*args, feat, lift)
    return wf.reshape(nq_pad, KNN), agg0


def _make_gred(ns_pad, c, nq_pad, op):
    chunk = nq_pad // NW
    K = KNN
    nv = c // L
    CB = min(nv, 16)
    nblk = nv // CB
    bq = max(1, min(4, chunk, 32768 // (K * c)))
    while chunk % bq:
        bq //= 2
    BK = bq * K
    NBAT = chunk // bq

    scratch = [
        pltpu.VMEM((chunk * K,), i32),
        pltpu.VMEM((BK, c), f32),
        pltpu.VMEM((BK, c), f32),
        pltpu.VMEM((chunk, c), f32),
        pltpu.SemaphoreType.DMA,
        pltpu.SemaphoreType.DMA,
    ]
    if op == "wsum":
        scratch.insert(1, pltpu.VMEM((chunk * K,), f32))

    def body(*refs):
        if op == "wsum":
            x_hbm, nbf_hbm, w_hbm, out_hbm, nb_v, w_v, r0, r1, o_v, s0, s1 = refs
        else:
            x_hbm, nbf_hbm, out_hbm, nb_v, r0, r1, o_v, s0, s1 = refs
        base = _wid() * chunk
        pltpu.sync_copy(nbf_hbm.at[pl.ds(base * K, chunk * K)], nb_v)
        if op == "wsum":
            pltpu.sync_copy(w_hbm.at[pl.ds(base * K, chunk * K)], w_v)

        def start(b, buf, sem):
            pltpu.async_copy(x_hbm.at[nb_v.at[pl.ds(b * BK, BK)]], buf, sem)

        def wait(buf, sem):
            pltpu.make_async_copy(x_hbm.at[nb_v.at[pl.ds(0, BK)]], buf,
                                  sem).wait()

        def compute(b, buf):
            def per_q(iq, carry):
                q = b * bq + iq
                r0k = iq * K

                def do_block(cb, carry2):
                    off = cb * (CB * L)
                    if op == "wsum":
                        acc = [jnp.zeros((L,), f32)] * CB
                        for k in range(K):
                            wk = plsc.load_gather(
                                w_v, [jnp.full((L,), q * K + k, i32)])
                            for j in range(CB):
                                acc[j] = acc[j] + (
                                    buf[r0k + k, pl.ds(off + j * L, L)] * wk)
                    else:
                        acc = [buf[r0k, pl.ds(off + j * L, L)]
                               for j in range(CB)]
                        for k in range(1, K):
                            for j in range(CB):
                                acc[j] = jnp.maximum(
                                    acc[j], buf[r0k + k, pl.ds(off + j * L, L)])
                    for j in range(CB):
                        o_v[q, pl.ds(off + j * L, L)] = acc[j]
                    return carry2

                if nblk == 1:
                    do_block(0, 0)
                else:
                    lax.fori_loop(0, nblk, do_block, 0)
                return carry

            if bq == 1:
                per_q(0, 0)
            else:
                lax.fori_loop(0, bq, per_q, 0)

        G2 = NBAT // 2
        start(0, r0, s0)

        def pair(g, carry):
            b0 = 2 * g
            start(b0 + 1, r1, s1)
            wait(r0, s0)
            compute(b0, r0)
            start(b0 + 2, r0, s0)
            wait(r1, s1)
            compute(b0 + 1, r1)
            return carry

        lax.fori_loop(0, G2 - 1, pair, 0)
        start(NBAT - 1, r1, s1)
        wait(r0, s0)
        compute(NBAT - 2, r0)
        wait(r1, s1)
        compute(NBAT - 1, r1)

        pltpu.sync_copy(o_v, out_hbm.at[pl.ds(base, chunk)])

    return pl.kernel(
        body, out_type=jax.ShapeDtypeStruct((nq_pad, c), f32),
        mesh=_mesh(), scratch_types=scratch,
        compiler_params=pltpu.CompilerParams(
            needs_layout_passes=False, use_tc_tiling_on_sc=False))


def _sc_wagg(x, nb, w):
    k = _make_gred(x.shape[0], x.shape[1], nb.shape[0], "wsum")
    return k(x, nb.reshape(-1), w.reshape(-1))


def _sc_maxgather(x, nb):
    k = _make_gred(x.shape[0], x.shape[1], nb.shape[0], "max")
    return k(x, nb.reshape(-1))


def _make_rowgather(ns_pad, c, nq_pad):
    chunk = nq_pad // NW

    def body(x_hbm, up_hbm, out_hbm, idx_v, rows_v, sem):
        base = _wid() * chunk
        pltpu.sync_copy(up_hbm.at[pl.ds(base, chunk)], idx_v)
        pltpu.async_copy(x_hbm.at[idx_v], rows_v, sem).wait()
        pltpu.sync_copy(rows_v, out_hbm.at[pl.ds(base, chunk)])

    return pl.kernel(
        body, out_type=jax.ShapeDtypeStruct((nq_pad, c), f32),
        mesh=_mesh(),
        scratch_types=[
            pltpu.VMEM((chunk,), i32),
            pltpu.VMEM((chunk, c), f32),
            pltpu.SemaphoreType.DMA,
        ],
        compiler_params=pltpu.CompilerParams(
            needs_layout_passes=False, use_tc_tiling_on_sc=False))


def _sc_rowgather(x, up):
    return _make_rowgather(x.shape[0], x.shape[1], up.shape[0])(x, up)


_PREC = lax.Precision.DEFAULT


def _dot(a, b):
    return jnp.dot(a, b, preferred_element_type=f32, precision=_PREC)


def _bn(n):
    return 256 if n % 256 == 0 else n


def _mm(x, W, b, relu=False, x2=None, W2=None):
    n, ci = x.shape
    co = W.shape[1]
    bn = _bn(n)
    grid = (n // bn,)
    b2 = b.reshape(1, co)
    dual = x2 is not None

    def body(*refs):
        if dual:
            x_ref, w_ref, b_ref, x2_ref, w2_ref, o_ref = refs
        else:
            x_ref, w_ref, b_ref, o_ref = refs
        acc = _dot(x_ref[...], w_ref[...]) + b_ref[...]
        if dual:
            acc = acc + _dot(x2_ref[...], w2_ref[...])
        if relu:
            acc = jnp.maximum(acc, 0.0)
        o_ref[...] = acc

    in_specs = [
        pl.BlockSpec((bn, ci), lambda i: (i, 0)),
        pl.BlockSpec((ci, co), lambda i: (0, 0)),
        pl.BlockSpec((1, co), lambda i: (0, 0)),
    ]
    args = [x, W, b2]
    if dual:
        ci2 = x2.shape[1]
        in_specs += [
            pl.BlockSpec((bn, ci2), lambda i: (i, 0)),
            pl.BlockSpec((ci2, co), lambda i: (0, 0)),
        ]
        args += [x2, W2]

    return pl.pallas_call(
        body, grid=grid,
        in_specs=in_specs,
        out_specs=pl.BlockSpec((bn, co), lambda i: (i, 0)),
        out_shape=jax.ShapeDtypeStruct((n, co), f32),
        compiler_params=pltpu.CompilerParams(
            dimension_semantics=("parallel",)),
    )(*args)


def _mmtail(ha, Wc, bc, Wu, bu, sc, Wsc):
    n, m = ha.shape
    o = Wu.shape[1]
    ci2 = sc.shape[1]
    bn = _bn(n)
    grid = (n // bn,)

    def body(ha_ref, wc_ref, bc_ref, wu_ref, bu_ref, sc_ref, wsc_ref, o_ref):
        h2 = jnp.maximum(_dot(ha_ref[...], wc_ref[...]) + bc_ref[...], 0.0)
        acc = _dot(h2, wu_ref[...]) + bu_ref[...]
        acc = acc + _dot(sc_ref[...], wsc_ref[...])
        o_ref[...] = jnp.maximum(acc, 0.0)

    return pl.pallas_call(
        body, grid=grid,
        in_specs=[
            pl.BlockSpec((bn, m), lambda i: (i, 0)),
            pl.BlockSpec((m, m), lambda i: (0, 0)),
            pl.BlockSpec((1, m), lambda i: (0, 0)),
            pl.BlockSpec((m, o), lambda i: (0, 0)),
            pl.BlockSpec((1, o), lambda i: (0, 0)),
            pl.BlockSpec((bn, ci2), lambda i: (i, 0)),
            pl.BlockSpec((ci2, o), lambda i: (0, 0)),
        ],
        out_specs=pl.BlockSpec((bn, o), lambda i: (i, 0)),
        out_shape=jax.ShapeDtypeStruct((n, o), f32),
        compiler_params=pltpu.CompilerParams(
            dimension_semantics=("parallel",)),
    )(ha, Wc, bc.reshape(1, m), Wu, bu.reshape(1, o), sc, Wsc)


def _pad_rows(a, P):
    return jnp.pad(a, ((0, P - a.shape[0]),) + ((0, 0),) * (a.ndim - 1))


def _pad_idx(a, P, ns):
    a = a.astype(i32)
    npad = P - a.shape[0]
    if npad == 0:
        return a
    K2 = a.shape[1]
    pad = (jnp.arange(npad * K2, dtype=i32) * 7919) % ns
    return jnp.concatenate([a, pad.reshape(npad, K2)], axis=0)


def _resnet(x, nb, w, p, strided):
    h = _mm(x, p["Wd"], p["bd"], relu=True)
    ha = _sc_wagg(h, nb, w)
    sc = _sc_maxgather(x, nb) if strided else x
    return _mmtail(ha, p["Wc"], p["bc"], p["Wu"], p["bu"], sc, p["Wsc"])


def kernel(feats, points_0, points_1, points_2, points_3,
           neighbors_0, neighbors_1, neighbors_2, neighbors_3,
           subsampling_0, subsampling_1, subsampling_2,
           upsampling_0, upsampling_1, upsampling_2, params):
    pts = [points_0, points_1, points_2, points_3]
    nbs = [neighbors_0, neighbors_1, neighbors_2, neighbors_3]
    subs = [subsampling_0, subsampling_1, subsampling_2]
    n = [p.shape[0] for p in pts]
    Pw = [-(-m // 512) * 512 for m in n]
    Pg = [-(-m // 256) * 256 for m in n]

    ptT = [_pad_rows(p, Pw[i]).T for i, p in enumerate(pts)]
    nbw = [_pad_idx(a, Pw[i], n[i]) for i, a in enumerate(nbs)]
    sbw = [_pad_idx(a, Pw[i + 1], n[i]) for i, a in enumerate(subs)]
    nbg = [a[:Pg[i]] for i, a in enumerate(nbw)]
    sbg = [a[:Pg[i + 1]] for i, a in enumerate(sbw)]
    u1 = _pad_idx(upsampling_1, Pg[1], n[2])[:, 0]
    u2 = _pad_idx(upsampling_2, Pg[2], n[3])[:, 0]

    pr = params
    featp = _pad_rows(feats, Pw[0])[:, 0]
    lift = jnp.zeros((L,), f32).at[0].set(pr["lift_W"][0, 0]).at[1].set(
        pr["lift_b"][0])

    s = 2.0
    w0, agg0 = _sc_weights(ptT[0], ptT[0], nbw[0].reshape(-1), s, featp, lift)
    a8 = jnp.pad(agg0[:Pg[0], None], ((0, 0), (0, 7)))
    W8 = jnp.pad(pr["s1_W"], ((0, 7), (0, 0)))
    x1 = _mm(a8, W8, pr["s1_b"], relu=True)
    x1 = _resnet(x1, nbg[0], w0[:Pg[0]], pr["e1_2"], False)

    ws0 = _sc_weights(ptT[0], ptT[1], sbw[0].reshape(-1), s)
    x2 = _resnet(x1, sbg[0], ws0[:Pg[1]], pr["e2_1"], True)
    w1 = _sc_weights(ptT[1], ptT[1], nbw[1].reshape(-1), 2 * s)[:Pg[1]]
    x2 = _resnet(x2, nbg[1], w1, pr["e2_2"], False)
    x2 = _resnet(x2, nbg[1], w1, pr["e2_3"], False)
    h = _mm(x2, pr["inv2_W"], pr["inv2_b"], relu=True)
    x2_inv = _sc_maxgather(h, nbg[1])

    ws1 = _sc_weights(ptT[1], ptT[2], sbw[1].reshape(-1), 2 * s)
    x3 = _resnet(x2, sbg[1], ws1[:Pg[2]], pr["e3_1"], True)
    w2 = _sc_weights(ptT[2], ptT[2], nbw[2].reshape(-1), 4 * s)[:Pg[2]]
    x3 = _resnet(x3, nbg[2], w2, pr["e3_2"], False)
    x3 = _resnet(x3, nbg[2], w2, pr["e3_3"], False)
    h = _mm(x3, pr["inv3_W"], pr["inv3_b"], relu=True)
    x3_inv = _sc_maxgather(h, nbg[2])

    ws2 = _sc_weights(ptT[2], ptT[3], sbw[2].reshape(-1), 4 * s)
    x4 = _resnet(x3, sbg[2], ws2[:Pg[3]], pr["e4_1"], True)
    w3 = _sc_weights(ptT[3], ptT[3], nbw[3].reshape(-1), 8 * s)[:Pg[3]]
    x4 = _resnet(x4, nbg[3], w3, pr["e4_2"], False)
    x4 = _resnet(x4, nbg[3], w3, pr["e4_3"], False)
    h = _mm(x4, pr["inv4_W"], pr["inv4_b"], relu=True)
    l4 = _sc_maxgather(h, nbg[3])

    d = l4.shape[1]
    l3g = _sc_rowgather(l4, u2)
    l3 = _mm(l3g, pr["dec3_W"][:d], pr["dec3_b"], relu=True,
             x2=x3_inv, W2=pr["dec3_W"][d:])
    l2g = _sc_rowgather(l3, u1)
    d3 = l3.shape[1]
    l2 = _mm(l2g, pr["dec2_W"][:d3], pr["dec2_b"], relu=False,
             x2=x2_inv, W2=pr["dec2_W"][d3:])

    return (l2[:n[1]], l3[:n[2]], l4[:n[3]])


# speedup vs baseline: 1.0904x; 1.0904x over previous
import functools

import jax
import jax.numpy as jnp
from jax import lax
from jax.experimental import pallas as pl
from jax.experimental.pallas import tpu as pltpu
from jax.experimental.pallas import tpu_sc as plsc

NC = 2
NS = 16
NW = NC * NS
L = 16
KNN = 32

f32 = jnp.float32
i32 = jnp.int32


def _mesh():
    return plsc.VectorSubcoreMesh(core_axis_name="c", subcore_axis_name="s")


def _wid():
    return lax.axis_index("s") * NC + lax.axis_index("c")


def _make_weights(ns_pad, nq_pad, sigma, with_feat):
    chunk = nq_pad // NW
    G = chunk // L
    neg_inv = -1.0 / (2.0 * sigma * sigma)
    K = KNN

    out_type = [jax.ShapeDtypeStruct((nq_pad * K,), f32)]
    if with_feat:
        out_type.append(jax.ShapeDtypeStruct((nq_pad,), f32))
    scratch = [
        pltpu.VMEM((ns_pad,), f32),
        pltpu.VMEM((ns_pad,), f32),
        pltpu.VMEM((ns_pad,), f32),
        pltpu.VMEM((chunk,), f32),
        pltpu.VMEM((chunk,), f32),
        pltpu.VMEM((chunk,), f32),
        pltpu.VMEM((chunk * K,), i32),
        pltpu.VMEM((K, chunk), f32),
        pltpu.VMEM((chunk * K,), f32),
    ]
    if with_feat:
        scratch += [
            pltpu.VMEM((ns_pad,), f32),
            pltpu.VMEM((chunk,), f32),
            pltpu.VMEM((L,), f32),
        ]

    def body(*refs):
        if with_feat:
            (spx, spy, spz, qpx, qpy, qpz, nbf, feat, lift, wout, aout,
             sx_v, sy_v, sz_v, qx_v, qy_v, qz_v, nb_v, w_v, wr_v,
             f_v, a_v, l_v) = refs
        else:
            (spx, spy, spz, qpx, qpy, qpz, nbf, wout,
             sx_v, sy_v, sz_v, qx_v, qy_v, qz_v, nb_v, w_v, wr_v) = refs
        wid = _wid()
        base = wid * chunk
        pltpu.sync_copy(spx, sx_v)
        pltpu.sync_copy(spy, sy_v)
        pltpu.sync_copy(spz, sz_v)
        pltpu.sync_copy(qpx.at[pl.ds(base, chunk)], qx_v)
        pltpu.sync_copy(qpy.at[pl.ds(base, chunk)], qy_v)
        pltpu.sync_copy(qpz.at[pl.ds(base, chunk)], qz_v)
        pltpu.sync_copy(nbf.at[pl.ds(base * K, chunk * K)], nb_v)
        if with_feat:
            pltpu.sync_copy(feat, f_v)
            pltpu.sync_copy(lift, l_v)

        lanes = jnp.arange(L, dtype=i32)

        def group(g, carry):
            off = g * L
            qx = qx_v[pl.ds(off, L)]
            qy = qy_v[pl.ds(off, L)]
            qz = qz_v[pl.ds(off, L)]
            pos0 = (lanes + off) * K
            acc = jnp.zeros((L,), f32)
            facc = jnp.zeros((L,), f32)
            for k in range(K):
                idx = plsc.load_gather(nb_v, [pos0 + k])
                sx = plsc.load_gather(sx_v, [idx])
                sy = plsc.load_gather(sy_v, [idx])
                sz = plsc.load_gather(sz_v, [idx])
                dx = sx - qx
                dy = sy - qy
                dz = sz - qz
                w = jnp.exp((dx * dx + dy * dy + dz * dz) * neg_inv)
                w_v[k, pl.ds(off, L)] = w
                acc = acc + w
                if with_feat:
                    facc = facc + w * plsc.load_gather(f_v, [idx])
            inv = 1.0 / (acc + 1e-8)
            for k in range(K):
                plsc.store_scatter(wr_v, [pos0 + k],
                                   w_v[k, pl.ds(off, L)] * inv)
            if with_feat:
                lv = l_v[pl.ds(0, L)]
                a_v[pl.ds(off, L)] = (lv[0] * facc + lv[1] * acc) * inv
            return carry

        lax.fori_loop(0, G, group, 0)
        pltpu.sync_copy(wr_v, wout.at[pl.ds(base * K, chunk * K)])
        if with_feat:
            pltpu.sync_copy(a_v, aout.at[pl.ds(base, chunk)])

    return pl.kernel(body, out_type=out_type, mesh=_mesh(),
                     scratch_types=scratch,
                     compiler_params=pltpu.CompilerParams(
                         needs_layout_passes=False))


def _sc_weights(spT, qpT, nbf, sigma, feat=None, lift=None):
    ns_pad = spT.shape[1]
    nq_pad = nbf.shape[0] // KNN
    k = _make_weights(ns_pad, nq_pad, sigma, feat is not None)
    args = (spT[0], spT[1], spT[2], qpT[0], qpT[1], qpT[2], nbf)
    if feat is None:
        (wf,) = k(*args)
        return wf.reshape(nq_pad, KNN)
    wf, agg0 = k(*args, feat, lift)
    return wf.reshape(nq_pad, KNN), agg0


def _make_gred(ns_pad, c, nq_pad, op, dtype=f32):
    chunk = nq_pad // NW
    K = KNN
    esz = 2 if dtype == jnp.bfloat16 else 4
    LV = 32 if esz == 2 else L
    nv = c // LV
    CB = min(nv, 16)
    nblk = nv // CB
    bq = max(1, min(4, chunk, 131072 // (K * c * esz)))
    while chunk % bq:
        bq //= 2
    BK = bq * K
    NBAT = chunk // bq

    scratch = [
        pltpu.VMEM((chunk * K,), i32),
        pltpu.VMEM((BK, c), dtype),
        pltpu.VMEM((BK, c), dtype),
        pltpu.VMEM((chunk, c), dtype),
        pltpu.SemaphoreType.DMA,
        pltpu.SemaphoreType.DMA,
    ]
    if op == "wsum":
        scratch.insert(1, pltpu.VMEM((chunk * K,), f32))

    def body(*refs):
        if op == "wsum":
            x_hbm, nbf_hbm, w_hbm, out_hbm, nb_v, w_v, r0, r1, o_v, s0, s1 = refs
        else:
            x_hbm, nbf_hbm, out_hbm, nb_v, r0, r1, o_v, s0, s1 = refs
        base = _wid() * chunk
        pltpu.sync_copy(nbf_hbm.at[pl.ds(base * K, chunk * K)], nb_v)
        if op == "wsum":
            pltpu.sync_copy(w_hbm.at[pl.ds(base * K, chunk * K)], w_v)

        def start(b, buf, sem):
            pltpu.async_copy(x_hbm.at[nb_v.at[pl.ds(b * BK, BK)]], buf, sem)

        def wait(buf, sem):
            pltpu.make_async_copy(x_hbm.at[nb_v.at[pl.ds(0, BK)]], buf,
                                  sem).wait()

        def compute(b, buf):
            def per_q(iq, carry):
                q = b * bq + iq
                r0k = iq * K

                def do_block(cb, carry2):
                    off = cb * (CB * LV)
                    if op == "wsum":
                        acc = [jnp.zeros((L,), f32)] * CB
                        for k in range(K):
                            wk = plsc.load_gather(
                                w_v, [jnp.full((L,), q * K + k, i32)])
                            for j in range(CB):
                                acc[j] = acc[j] + (
                                    buf[r0k + k, pl.ds(off + j * L, L)] * wk)
                    else:
                        acc = [buf[r0k, pl.ds(off + j * LV, LV)]
                               for j in range(CB)]
                        for k in range(1, K):
                            for j in range(CB):
                                acc[j] = jnp.maximum(
                                    acc[j],
                                    buf[r0k + k, pl.ds(off + j * LV, LV)])
                    for j in range(CB):
                        o_v[q, pl.ds(off + j * LV, LV)] = acc[j]
                    return carry2

                if nblk == 1:
                    do_block(0, 0)
                else:
                    lax.fori_loop(0, nblk, do_block, 0)
                return carry

            if bq == 1:
                per_q(0, 0)
            else:
                lax.fori_loop(0, bq, per_q, 0)

        G2 = NBAT // 2
        start(0, r0, s0)

        def pair(g, carry):
            b0 = 2 * g
            start(b0 + 1, r1, s1)
            wait(r0, s0)
            compute(b0, r0)
            start(b0 + 2, r0, s0)
            wait(r1, s1)
            compute(b0 + 1, r1)
            return carry

        lax.fori_loop(0, G2 - 1, pair, 0)
        start(NBAT - 1, r1, s1)
        wait(r0, s0)
        compute(NBAT - 2, r0)
        wait(r1, s1)
        compute(NBAT - 1, r1)

        pltpu.sync_copy(o_v, out_hbm.at[pl.ds(base, chunk)])

    return pl.kernel(
        body, out_type=jax.ShapeDtypeStruct((nq_pad, c), dtype),
        mesh=_mesh(), scratch_types=scratch,
        compiler_params=pltpu.CompilerParams(
            needs_layout_passes=False, use_tc_tiling_on_sc=False))


def _sc_wagg(x, nb, w):
    k = _make_gred(x.shape[0], x.shape[1], nb.shape[0], "wsum")
    return k(x, nb.reshape(-1), w.reshape(-1))


def _sc_maxgather(x, nb):
    k = _make_gred(x.shape[0], x.shape[1], nb.shape[0], "max", x.dtype)
    return k(x, nb.reshape(-1))


def _make_rowgather(ns_pad, c, nq_pad):
    chunk = nq_pad // NW

    def body(x_hbm, up_hbm, out_hbm, idx_v, rows_v, sem):
        base = _wid() * chunk
        pltpu.sync_copy(up_hbm.at[pl.ds(base, chunk)], idx_v)
        pltpu.async_copy(x_hbm.at[idx_v], rows_v, sem).wait()
        pltpu.sync_copy(rows_v, out_hbm.at[pl.ds(base, chunk)])

    return pl.kernel(
        body, out_type=jax.ShapeDtypeStruct((nq_pad, c), f32),
        mesh=_mesh(),
        scratch_types=[
            pltpu.VMEM((chunk,), i32),
            pltpu.VMEM((chunk, c), f32),
            pltpu.SemaphoreType.DMA,
        ],
        compiler_params=pltpu.CompilerParams(
            needs_layout_passes=False, use_tc_tiling_on_sc=False))


def _sc_rowgather(x, up):
    return _make_rowgather(x.shape[0], x.shape[1], up.shape[0])(x, up)


_PREC = lax.Precision.DEFAULT


def _dot(a, b):
    return jnp.dot(a, b, preferred_element_type=f32, precision=_PREC)


def _bn(n):
    return 256 if n % 256 == 0 else n


def _mm(x, W, b, relu=False, x2=None, W2=None, out_dtype=f32):
    n, ci = x.shape
    co = W.shape[1]
    bn = _bn(n)
    grid = (n // bn,)
    b2 = b.reshape(1, co)
    dual = x2 is not None

    def body(*refs):
        if dual:
            x_ref, w_ref, b_ref, x2_ref, w2_ref, o_ref = refs
        else:
            x_ref, w_ref, b_ref, o_ref = refs
        acc = _dot(x_ref[...], w_ref[...]) + b_ref[...]
        if dual:
            acc = acc + _dot(x2_ref[...].astype(f32), w2_ref[...])
        if relu:
            acc = jnp.maximum(acc, 0.0)
        o_ref[...] = acc.astype(out_dtype)

    in_specs = [
        pl.BlockSpec((bn, ci), lambda i: (i, 0)),
        pl.BlockSpec((ci, co), lambda i: (0, 0)),
        pl.BlockSpec((1, co), lambda i: (0, 0)),
    ]
    args = [x, W, b2]
    if dual:
        ci2 = x2.shape[1]
        in_specs += [
            pl.BlockSpec((bn, ci2), lambda i: (i, 0)),
            pl.BlockSpec((ci2, co), lambda i: (0, 0)),
        ]
        args += [x2, W2]

    return pl.pallas_call(
        body, grid=grid,
        in_specs=in_specs,
        out_specs=pl.BlockSpec((bn, co), lambda i: (i, 0)),
        out_shape=jax.ShapeDtypeStruct((n, co), out_dtype),
        compiler_params=pltpu.CompilerParams(
            dimension_semantics=("parallel",)),
    )(*args)


def _mmtail(ha, Wc, bc, Wu, bu, sc, Wsc):
    n, m = ha.shape
    o = Wu.shape[1]
    ci2 = sc.shape[1]
    bn = _bn(n)
    grid = (n // bn,)

    def body(ha_ref, wc_ref, bc_ref, wu_ref, bu_ref, sc_ref, wsc_ref, o_ref):
        h2 = jnp.maximum(_dot(ha_ref[...], wc_ref[...]) + bc_ref[...], 0.0)
        acc = _dot(h2, wu_ref[...]) + bu_ref[...]
        acc = acc + _dot(sc_ref[...], wsc_ref[...])
        o_ref[...] = jnp.maximum(acc, 0.0)

    return pl.pallas_call(
        body, grid=grid,
        in_specs=[
            pl.BlockSpec((bn, m), lambda i: (i, 0)),
            pl.BlockSpec((m, m), lambda i: (0, 0)),
            pl.BlockSpec((1, m), lambda i: (0, 0)),
            pl.BlockSpec((m, o), lambda i: (0, 0)),
            pl.BlockSpec((1, o), lambda i: (0, 0)),
            pl.BlockSpec((bn, ci2), lambda i: (i, 0)),
            pl.BlockSpec((ci2, o), lambda i: (0, 0)),
        ],
        out_specs=pl.BlockSpec((bn, o), lambda i: (i, 0)),
        out_shape=jax.ShapeDtypeStruct((n, o), f32),
        compiler_params=pltpu.CompilerParams(
            dimension_semantics=("parallel",)),
    )(ha, Wc, bc.reshape(1, m), Wu, bu.reshape(1, o), sc, Wsc)


def _pad_rows(a, P):
    return jnp.pad(a, ((0, P - a.shape[0]),) + ((0, 0),) * (a.ndim - 1))


def _pad_idx(a, P, ns):
    a = a.astype(i32)
    npad = P - a.shape[0]
    if npad == 0:
        return a
    K2 = a.shape[1]
    pad = (jnp.arange(npad * K2, dtype=i32) * 7919) % ns
    return jnp.concatenate([a, pad.reshape(npad, K2)], axis=0)


def _resnet(x, nb, w, p, strided):
    h = _mm(x, p["Wd"], p["bd"], relu=True)
    ha = _sc_wagg(h, nb, w)
    sc = _sc_maxgather(x, nb) if strided else x
    return _mmtail(ha, p["Wc"], p["bc"], p["Wu"], p["bu"], sc, p["Wsc"])


def kernel(feats, points_0, points_1, points_2, points_3,
           neighbors_0, neighbors_1, neighbors_2, neighbors_3,
           subsampling_0, subsampling_1, subsampling_2,
           upsampling_0, upsampling_1, upsampling_2, params):
    pts = [points_0, points_1, points_2, points_3]
    nbs = [neighbors_0, neighbors_1, neighbors_2, neighbors_3]
    subs = [subsampling_0, subsampling_1, subsampling_2]
    n = [p.shape[0] for p in pts]
    Pw = [-(-m // 512) * 512 for m in n]
    Pg = [-(-m // 256) * 256 for m in n]

    ptT = [_pad_rows(p, Pw[i]).T for i, p in enumerate(pts)]
    nbw = [_pad_idx(a, Pw[i], n[i]) for i, a in enumerate(nbs)]
    sbw = [_pad_idx(a, Pw[i + 1], n[i]) for i, a in enumerate(subs)]
    nbg = [a[:Pg[i]] for i, a in enumerate(nbw)]
    sbg = [a[:Pg[i + 1]] for i, a in enumerate(sbw)]
    u1 = _pad_idx(upsampling_1, Pg[1], n[2])[:, 0]
    u2 = _pad_idx(upsampling_2, Pg[2], n[3])[:, 0]

    pr = params
    featp = _pad_rows(feats, Pw[0])[:, 0]
    lift = jnp.zeros((L,), f32).at[0].set(pr["lift_W"][0, 0]).at[1].set(
        pr["lift_b"][0])

    s = 2.0
    w0, agg0 = _sc_weights(ptT[0], ptT[0], nbw[0].reshape(-1), s, featp, lift)
    a8 = jnp.pad(agg0[:Pg[0], None], ((0, 0), (0, 7)))
    W8 = jnp.pad(pr["s1_W"], ((0, 7), (0, 0)))
    x1 = _mm(a8, W8, pr["s1_b"], relu=True)
    x1 = _resnet(x1, nbg[0], w0[:Pg[0]], pr["e1_2"], False)

    ws0 = _sc_weights(ptT[0], ptT[1], sbw[0].reshape(-1), s)
    x2 = _resnet(x1, sbg[0], ws0[:Pg[1]], pr["e2_1"], True)
    w1 = _sc_weights(ptT[1], ptT[1], nbw[1].reshape(-1), 2 * s)[:Pg[1]]
    x2 = _resnet(x2, nbg[1], w1, pr["e2_2"], False)
    x2 = _resnet(x2, nbg[1], w1, pr["e2_3"], False)
    h = _mm(x2, pr["inv2_W"], pr["inv2_b"], relu=True,
            out_dtype=jnp.bfloat16)
    x2_inv = _sc_maxgather(h, nbg[1])

    ws1 = _sc_weights(ptT[1], ptT[2], sbw[1].reshape(-1), 2 * s)
    x3 = _resnet(x2, sbg[1], ws1[:Pg[2]], pr["e3_1"], True)
    w2 = _sc_weights(ptT[2], ptT[2], nbw[2].reshape(-1), 4 * s)[:Pg[2]]
    x3 = _resnet(x3, nbg[2], w2, pr["e3_2"], False)
    x3 = _resnet(x3, nbg[2], w2, pr["e3_3"], False)
    h = _mm(x3, pr["inv3_W"], pr["inv3_b"], relu=True,
            out_dtype=jnp.bfloat16)
    x3_inv = _sc_maxgather(h, nbg[2])

    ws2 = _sc_weights(ptT[2], ptT[3], sbw[2].reshape(-1), 4 * s)
    x4 = _resnet(x3, sbg[2], ws2[:Pg[3]], pr["e4_1"], True)
    w3 = _sc_weights(ptT[3], ptT[3], nbw[3].reshape(-1), 8 * s)[:Pg[3]]
    x4 = _resnet(x4, nbg[3], w3, pr["e4_2"], False)
    x4 = _resnet(x4, nbg[3], w3, pr["e4_3"], False)
    h = _mm(x4, pr["inv4_W"], pr["inv4_b"], relu=True)
    l4 = _sc_maxgather(h, nbg[3])

    d = l4.shape[1]
    l3g = _sc_rowgather(l4, u2)
    l3 = _mm(l3g, pr["dec3_W"][:d], pr["dec3_b"], relu=True,
             x2=x3_inv, W2=pr["dec3_W"][d:])
    l2g = _sc_rowgather(l3, u1)
    d3 = l3.shape[1]
    l2 = _mm(l2g, pr["dec2_W"][:d3], pr["dec2_b"], relu=False,
             x2=x2_inv, W2=pr["dec2_W"][d3:])

    return (l2[:n[1]], l3[:n[2]], l4[:n[3]])


# speedup vs baseline: 1.1903x; 1.0916x over previous
import functools

import jax
import jax.numpy as jnp
from jax import lax
from jax.experimental import pallas as pl
from jax.experimental.pallas import tpu as pltpu
from jax.experimental.pallas import tpu_sc as plsc

NC = 2
NS = 16
NW = NC * NS
L = 16
KNN = 32

f32 = jnp.float32
i32 = jnp.int32


def _mesh():
    return plsc.VectorSubcoreMesh(core_axis_name="c", subcore_axis_name="s")


def _wid():
    return lax.axis_index("s") * NC + lax.axis_index("c")


def _make_weights(ns_pad, nq_pad, sigma, with_feat):
    chunk = nq_pad // NW
    G = chunk // L
    neg_inv = -1.0 / (2.0 * sigma * sigma)
    K = KNN

    out_type = [jax.ShapeDtypeStruct((nq_pad * K,), f32)]
    if with_feat:
        out_type.append(jax.ShapeDtypeStruct((nq_pad,), f32))
    scratch = [
        pltpu.VMEM((ns_pad,), f32),
        pltpu.VMEM((ns_pad,), f32),
        pltpu.VMEM((ns_pad,), f32),
        pltpu.VMEM((chunk,), f32),
        pltpu.VMEM((chunk,), f32),
        pltpu.VMEM((chunk,), f32),
        pltpu.VMEM((chunk * K,), i32),
        pltpu.VMEM((K, chunk), f32),
        pltpu.VMEM((chunk * K,), f32),
    ]
    if with_feat:
        scratch += [
            pltpu.VMEM((ns_pad,), f32),
            pltpu.VMEM((chunk,), f32),
            pltpu.VMEM((L,), f32),
        ]

    def body(*refs):
        if with_feat:
            (spx, spy, spz, qpx, qpy, qpz, nbf, feat, lift, wout, aout,
             sx_v, sy_v, sz_v, qx_v, qy_v, qz_v, nb_v, w_v, wr_v,
             f_v, a_v, l_v) = refs
        else:
            (spx, spy, spz, qpx, qpy, qpz, nbf, wout,
             sx_v, sy_v, sz_v, qx_v, qy_v, qz_v, nb_v, w_v, wr_v) = refs
        wid = _wid()
        base = wid * chunk
        pltpu.sync_copy(spx, sx_v)
        pltpu.sync_copy(spy, sy_v)
        pltpu.sync_copy(spz, sz_v)
        pltpu.sync_copy(qpx.at[pl.ds(base, chunk)], qx_v)
        pltpu.sync_copy(qpy.at[pl.ds(base, chunk)], qy_v)
        pltpu.sync_copy(qpz.at[pl.ds(base, chunk)], qz_v)
        pltpu.sync_copy(nbf.at[pl.ds(base * K, chunk * K)], nb_v)
        if with_feat:
            pltpu.sync_copy(feat, f_v)
            pltpu.sync_copy(lift, l_v)

        lanes = jnp.arange(L, dtype=i32)

        def group(g, carry):
            off = g * L
            qx = qx_v[pl.ds(off, L)]
            qy = qy_v[pl.ds(off, L)]
            qz = qz_v[pl.ds(off, L)]
            pos0 = (lanes + off) * K
            acc = jnp.zeros((L,), f32)
            facc = jnp.zeros((L,), f32)
            for k in range(K):
                idx = plsc.load_gather(nb_v, [pos0 + k])
                sx = plsc.load_gather(sx_v, [idx])
                sy = plsc.load_gather(sy_v, [idx])
                sz = plsc.load_gather(sz_v, [idx])
                dx = sx - qx
                dy = sy - qy
                dz = sz - qz
                w = jnp.exp((dx * dx + dy * dy + dz * dz) * neg_inv)
                w_v[k, pl.ds(off, L)] = w
                acc = acc + w
                if with_feat:
                    facc = facc + w * plsc.load_gather(f_v, [idx])
            inv = 1.0 / (acc + 1e-8)
            for k in range(K):
                plsc.store_scatter(wr_v, [pos0 + k],
                                   w_v[k, pl.ds(off, L)] * inv)
            if with_feat:
                lv = l_v[pl.ds(0, L)]
                a_v[pl.ds(off, L)] = (lv[0] * facc + lv[1] * acc) * inv
            return carry

        lax.fori_loop(0, G, group, 0)
        pltpu.sync_copy(wr_v, wout.at[pl.ds(base * K, chunk * K)])
        if with_feat:
            pltpu.sync_copy(a_v, aout.at[pl.ds(base, chunk)])

    return pl.kernel(body, out_type=out_type, mesh=_mesh(),
                     scratch_types=scratch,
                     compiler_params=pltpu.CompilerParams(
                         needs_layout_passes=False))


def _sc_weights(spT, qpT, nbf, sigma, feat=None, lift=None):
    ns_pad = spT.shape[1]
    nq_pad = nbf.shape[0] // KNN
    k = _make_weights(ns_pad, nq_pad, sigma, feat is not None)
    args = (spT[0], spT[1], spT[2], qpT[0], qpT[1], qpT[2], nbf)
    if feat is None:
        (wf,) = k(*args)
        return wf.reshape(nq_pad, KNN)
    wf, agg0 = k(*args, feat, lift)
    return wf.reshape(nq_pad, KNN), agg0


def _make_gred(ns_pad, c, nq_pad, op, dtype=f32):
    chunk = nq_pad // NW
    K = KNN
    esz = 2 if dtype == jnp.bfloat16 else 4
    LV = 32 if esz == 2 else L
    nv = c // LV
    CB = min(nv, 16)
    nblk = nv // CB
    bq = max(1, min(4, chunk, 131072 // (K * c * esz)))
    while chunk % bq:
        bq //= 2
    BK = bq * K
    NBAT = chunk // bq

    scratch = [
        pltpu.VMEM((chunk * K,), i32),
        pltpu.VMEM((BK, c), dtype),
        pltpu.VMEM((BK, c), dtype),
        pltpu.VMEM((chunk, c), dtype),
        pltpu.SemaphoreType.DMA,
        pltpu.SemaphoreType.DMA,
    ]
    if op == "wsum":
        scratch.insert(1, pltpu.VMEM((chunk * K,), f32))

    def body(*refs):
        if op == "wsum":
            x_hbm, nbf_hbm, w_hbm, out_hbm, nb_v, w_v, r0, r1, o_v, s0, s1 = refs
        else:
            x_hbm, nbf_hbm, out_hbm, nb_v, r0, r1, o_v, s0, s1 = refs
        base = _wid() * chunk
        pltpu.sync_copy(nbf_hbm.at[pl.ds(base * K, chunk * K)], nb_v)
        if op == "wsum":
            pltpu.sync_copy(w_hbm.at[pl.ds(base * K, chunk * K)], w_v)

        def start(b, buf, sem):
            pltpu.async_copy(x_hbm.at[nb_v.at[pl.ds(b * BK, BK)]], buf, sem)

        def wait(buf, sem):
            pltpu.make_async_copy(x_hbm.at[nb_v.at[pl.ds(0, BK)]], buf,
                                  sem).wait()

        def compute(b, buf):
            def per_q(iq, carry):
                q = b * bq + iq
                r0k = iq * K

                def do_block(cb, carry2):
                    off = cb * (CB * LV)
                    if op == "wsum":
                        acc = [jnp.zeros((L,), f32)] * CB
                        for k in range(K):
                            wk = plsc.load_gather(
                                w_v, [jnp.full((L,), q * K + k, i32)])
                            for j in range(CB):
                                acc[j] = acc[j] + (
                                    buf[r0k + k, pl.ds(off + j * L, L)] * wk)
                    else:
                        acc = [buf[r0k, pl.ds(off + j * LV, LV)]
                               for j in range(CB)]
                        for k in range(1, K):
                            for j in range(CB):
                                acc[j] = jnp.maximum(
                                    acc[j],
                                    buf[r0k + k, pl.ds(off + j * LV, LV)])
                    for j in range(CB):
                        o_v[q, pl.ds(off + j * LV, LV)] = acc[j]
                    return carry2

                if nblk == 1:
                    do_block(0, 0)
                else:
                    lax.fori_loop(0, nblk, do_block, 0)
                return carry

            if bq == 1:
                per_q(0, 0)
            else:
                lax.fori_loop(0, bq, per_q, 0)

        G2 = NBAT // 2
        start(0, r0, s0)

        def pair(g, carry):
            b0 = 2 * g
            start(b0 + 1, r1, s1)
            wait(r0, s0)
            compute(b0, r0)
            start(b0 + 2, r0, s0)
            wait(r1, s1)
            compute(b0 + 1, r1)
            return carry

        lax.fori_loop(0, G2 - 1, pair, 0)
        start(NBAT - 1, r1, s1)
        wait(r0, s0)
        compute(NBAT - 2, r0)
        wait(r1, s1)
        compute(NBAT - 1, r1)

        pltpu.sync_copy(o_v, out_hbm.at[pl.ds(base, chunk)])

    return pl.kernel(
        body, out_type=jax.ShapeDtypeStruct((nq_pad, c), dtype),
        mesh=_mesh(), scratch_types=scratch,
        compiler_params=pltpu.CompilerParams(
            needs_layout_passes=False, use_tc_tiling_on_sc=False))


def _sc_wagg(x, nb, w):
    k = _make_gred(x.shape[0], x.shape[1], nb.shape[0], "wsum")
    return k(x, nb.reshape(-1), w.reshape(-1))


def _sc_maxgather(x, nb):
    k = _make_gred(x.shape[0], x.shape[1], nb.shape[0], "max", x.dtype)
    return k(x, nb.reshape(-1))


def _make_rowgather(ns_pad, c, nq_pad, dtype=f32):
    chunk = nq_pad // NW

    def body(x_hbm, up_hbm, out_hbm, idx_v, rows_v, sem):
        base = _wid() * chunk
        pltpu.sync_copy(up_hbm.at[pl.ds(base, chunk)], idx_v)
        pltpu.async_copy(x_hbm.at[idx_v], rows_v, sem).wait()
        pltpu.sync_copy(rows_v, out_hbm.at[pl.ds(base, chunk)])

    return pl.kernel(
        body, out_type=jax.ShapeDtypeStruct((nq_pad, c), dtype),
        mesh=_mesh(),
        scratch_types=[
            pltpu.VMEM((chunk,), i32),
            pltpu.VMEM((chunk, c), dtype),
            pltpu.SemaphoreType.DMA,
        ],
        compiler_params=pltpu.CompilerParams(
            needs_layout_passes=False, use_tc_tiling_on_sc=False))


def _sc_rowgather(x, up):
    return _make_rowgather(x.shape[0], x.shape[1], up.shape[0], x.dtype)(x, up)


_PREC = lax.Precision.DEFAULT


def _dot(a, b):
    return jnp.dot(a, b, preferred_element_type=f32, precision=_PREC)


def _bn(n):
    return 256 if n % 256 == 0 else n


def _mm(x, W, b, relu=False, x2=None, W2=None, out_dtype=f32):
    n, ci = x.shape
    co = W.shape[1]
    bn = _bn(n)
    grid = (n // bn,)
    b2 = b.reshape(1, co)
    dual = x2 is not None

    def body(*refs):
        if dual:
            x_ref, w_ref, b_ref, x2_ref, w2_ref, o_ref = refs
        else:
            x_ref, w_ref, b_ref, o_ref = refs
        acc = _dot(x_ref[...].astype(f32), w_ref[...]) + b_ref[...]
        if dual:
            acc = acc + _dot(x2_ref[...].astype(f32), w2_ref[...])
        if relu:
            acc = jnp.maximum(acc, 0.0)
        o_ref[...] = acc.astype(out_dtype)

    in_specs = [
        pl.BlockSpec((bn, ci), lambda i: (i, 0)),
        pl.BlockSpec((ci, co), lambda i: (0, 0)),
        pl.BlockSpec((1, co), lambda i: (0, 0)),
    ]
    args = [x, W, b2]
    if dual:
        ci2 = x2.shape[1]
        in_specs += [
            pl.BlockSpec((bn, ci2), lambda i: (i, 0)),
            pl.BlockSpec((ci2, co), lambda i: (0, 0)),
        ]
        args += [x2, W2]

    return pl.pallas_call(
        body, grid=grid,
        in_specs=in_specs,
        out_specs=pl.BlockSpec((bn, co), lambda i: (i, 0)),
        out_shape=jax.ShapeDtypeStruct((n, co), out_dtype),
        compiler_params=pltpu.CompilerParams(
            dimension_semantics=("parallel",)),
    )(*args)


def _mmtail(ha, Wc, bc, Wu, bu, sc, Wsc):
    n, m = ha.shape
    o = Wu.shape[1]
    ci2 = sc.shape[1]
    bn = _bn(n)
    grid = (n // bn,)

    def body(ha_ref, wc_ref, bc_ref, wu_ref, bu_ref, sc_ref, wsc_ref, o_ref):
        h2 = jnp.maximum(_dot(ha_ref[...], wc_ref[...]) + bc_ref[...], 0.0)
        acc = _dot(h2, wu_ref[...]) + bu_ref[...]
        acc = acc + _dot(sc_ref[...].astype(f32), wsc_ref[...])
        o_ref[...] = jnp.maximum(acc, 0.0)

    return pl.pallas_call(
        body, grid=grid,
        in_specs=[
            pl.BlockSpec((bn, m), lambda i: (i, 0)),
            pl.BlockSpec((m, m), lambda i: (0, 0)),
            pl.BlockSpec((1, m), lambda i: (0, 0)),
            pl.BlockSpec((m, o), lambda i: (0, 0)),
            pl.BlockSpec((1, o), lambda i: (0, 0)),
            pl.BlockSpec((bn, ci2), lambda i: (i, 0)),
            pl.BlockSpec((ci2, o), lambda i: (0, 0)),
        ],
        out_specs=pl.BlockSpec((bn, o), lambda i: (i, 0)),
        out_shape=jax.ShapeDtypeStruct((n, o), f32),
        compiler_params=pltpu.CompilerParams(
            dimension_semantics=("parallel",)),
    )(ha, Wc, bc.reshape(1, m), Wu, bu.reshape(1, o), sc, Wsc)


def _pad_rows(a, P):
    return jnp.pad(a, ((0, P - a.shape[0]),) + ((0, 0),) * (a.ndim - 1))


def _pad_idx(a, P, ns):
    a = a.astype(i32)
    npad = P - a.shape[0]
    if npad == 0:
        return a
    K2 = a.shape[1]
    pad = (jnp.arange(npad * K2, dtype=i32) * 7919) % ns
    return jnp.concatenate([a, pad.reshape(npad, K2)], axis=0)


def _resnet(x, nb, w, p, strided):
    h = _mm(x, p["Wd"], p["bd"], relu=True)
    ha = _sc_wagg(h, nb, w)
    sc = _sc_maxgather(x.astype(jnp.bfloat16), nb) if strided else x
    return _mmtail(ha, p["Wc"], p["bc"], p["Wu"], p["bu"], sc, p["Wsc"])


def kernel(feats, points_0, points_1, points_2, points_3,
           neighbors_0, neighbors_1, neighbors_2, neighbors_3,
           subsampling_0, subsampling_1, subsampling_2,
           upsampling_0, upsampling_1, upsampling_2, params):
    pts = [points_0, points_1, points_2, points_3]
    nbs = [neighbors_0, neighbors_1, neighbors_2, neighbors_3]
    subs = [subsampling_0, subsampling_1, subsampling_2]
    n = [p.shape[0] for p in pts]
    Pw = [-(-m // 512) * 512 for m in n]
    Pg = [-(-m // 256) * 256 for m in n]

    ptT = [_pad_rows(p, Pw[i]).T for i, p in enumerate(pts)]
    nbw = [_pad_idx(a, Pw[i], n[i]) for i, a in enumerate(nbs)]
    sbw = [_pad_idx(a, Pw[i + 1], n[i]) for i, a in enumerate(subs)]
    nbg = [a[:Pg[i]] for i, a in enumerate(nbw)]
    sbg = [a[:Pg[i + 1]] for i, a in enumerate(sbw)]
    u1 = _pad_idx(upsampling_1, Pg[1], n[2])[:, 0]
    u2 = _pad_idx(upsampling_2, Pg[2], n[3])[:, 0]

    pr = params
    featp = _pad_rows(feats, Pw[0])[:, 0]
    lift = jnp.zeros((L,), f32).at[0].set(pr["lift_W"][0, 0]).at[1].set(
        pr["lift_b"][0])

    s = 2.0
    w0, agg0 = _sc_weights(ptT[0], ptT[0], nbw[0].reshape(-1), s, featp, lift)
    a8 = jnp.pad(agg0[:Pg[0], None], ((0, 0), (0, 7)))
    W8 = jnp.pad(pr["s1_W"], ((0, 7), (0, 0)))
    x1 = _mm(a8, W8, pr["s1_b"], relu=True)
    x1 = _resnet(x1, nbg[0], w0[:Pg[0]], pr["e1_2"], False)

    ws0 = _sc_weights(ptT[0], ptT[1], sbw[0].reshape(-1), s)
    x2 = _resnet(x1, sbg[0], ws0[:Pg[1]], pr["e2_1"], True)
    w1 = _sc_weights(ptT[1], ptT[1], nbw[1].reshape(-1), 2 * s)[:Pg[1]]
    x2 = _resnet(x2, nbg[1], w1, pr["e2_2"], False)
    x2 = _resnet(x2, nbg[1], w1, pr["e2_3"], False)
    h = _mm(x2, pr["inv2_W"], pr["inv2_b"], relu=True,
            out_dtype=jnp.bfloat16)
    x2_inv = _sc_maxgather(h, nbg[1])

    ws1 = _sc_weights(ptT[1], ptT[2], sbw[1].reshape(-1), 2 * s)
    x3 = _resnet(x2, sbg[1], ws1[:Pg[2]], pr["e3_1"], True)
    w2 = _sc_weights(ptT[2], ptT[2], nbw[2].reshape(-1), 4 * s)[:Pg[2]]
    x3 = _resnet(x3, nbg[2], w2, pr["e3_2"], False)
    x3 = _resnet(x3, nbg[2], w2, pr["e3_3"], False)
    h = _mm(x3, pr["inv3_W"], pr["inv3_b"], relu=True,
            out_dtype=jnp.bfloat16)
    x3_inv = _sc_maxgather(h, nbg[2])

    ws2 = _sc_weights(ptT[2], ptT[3], sbw[2].reshape(-1), 4 * s)
    x4 = _resnet(x3, sbg[2], ws2[:Pg[3]], pr["e4_1"], True)
    w3 = _sc_weights(ptT[3], ptT[3], nbw[3].reshape(-1), 8 * s)[:Pg[3]]
    x4 = _resnet(x4, nbg[3], w3, pr["e4_2"], False)
    x4 = _resnet(x4, nbg[3], w3, pr["e4_3"], False)
    h = _mm(x4, pr["inv4_W"], pr["inv4_b"], relu=True,
            out_dtype=jnp.bfloat16)
    l4b = _sc_maxgather(h, nbg[3])
    l4 = l4b.astype(f32)

    d = l4.shape[1]
    l3g = _sc_rowgather(l4b, u2)
    l3 = _mm(l3g, pr["dec3_W"][:d], pr["dec3_b"], relu=True,
             x2=x3_inv, W2=pr["dec3_W"][d:])
    l2g = _sc_rowgather(l3, u1)
    d3 = l3.shape[1]
    l2 = _mm(l2g, pr["dec2_W"][:d3], pr["dec2_b"], relu=False,
             x2=x2_inv, W2=pr["dec2_W"][d3:])

    return (l2[:n[1]], l3[:n[2]], l4[:n[3]])


# speedup vs baseline: 1.1948x; 1.0037x over previous
import jax
import jax.numpy as jnp
from jax import lax
from jax.experimental import pallas as pl
from jax.experimental.pallas import tpu as pltpu
from jax.experimental.pallas import tpu_sc as plsc

NC = 2
NS = 16
NW = NC * NS
L = 16
KNN = 32

f32 = jnp.float32
i32 = jnp.int32


def _mesh():
    return plsc.VectorSubcoreMesh(core_axis_name="c", subcore_axis_name="s")


def _wid():
    return lax.axis_index("s") * NC + lax.axis_index("c")


def _make_weights(ns_pad, nq_pad, sigma, with_feat):
    chunk = nq_pad // NW
    G = chunk // L
    neg_inv = -1.0 / (2.0 * sigma * sigma)
    K = KNN

    out_type = [jax.ShapeDtypeStruct((nq_pad * K,), f32)]
    if with_feat:
        out_type.append(jax.ShapeDtypeStruct((nq_pad,), f32))
    scratch = [
        pltpu.VMEM((ns_pad,), f32),
        pltpu.VMEM((ns_pad,), f32),
        pltpu.VMEM((ns_pad,), f32),
        pltpu.VMEM((chunk,), f32),
        pltpu.VMEM((chunk,), f32),
        pltpu.VMEM((chunk,), f32),
        pltpu.VMEM((chunk * K,), i32),
        pltpu.VMEM((K, chunk), f32),
        pltpu.VMEM((chunk * K,), f32),
    ]
    if with_feat:
        scratch += [
            pltpu.VMEM((ns_pad,), f32),
            pltpu.VMEM((chunk,), f32),
            pltpu.VMEM((L,), f32),
        ]

    def body(*refs):
        if with_feat:
            (spx, spy, spz, qpx, qpy, qpz, nbf, feat, lift, wout, aout,
             sx_v, sy_v, sz_v, qx_v, qy_v, qz_v, nb_v, w_v, wr_v,
             f_v, a_v, l_v) = refs
        else:
            (spx, spy, spz, qpx, qpy, qpz, nbf, wout,
             sx_v, sy_v, sz_v, qx_v, qy_v, qz_v, nb_v, w_v, wr_v) = refs
        wid = _wid()
        base = wid * chunk
        pltpu.sync_copy(spx, sx_v)
        pltpu.sync_copy(spy, sy_v)
        pltpu.sync_copy(spz, sz_v)
        pltpu.sync_copy(qpx.at[pl.ds(base, chunk)], qx_v)
        pltpu.sync_copy(qpy.at[pl.ds(base, chunk)], qy_v)
        pltpu.sync_copy(qpz.at[pl.ds(base, chunk)], qz_v)
        pltpu.sync_copy(nbf.at[pl.ds(base * K, chunk * K)], nb_v)
        if with_feat:
            pltpu.sync_copy(feat, f_v)
            pltpu.sync_copy(lift, l_v)

        lanes = jnp.arange(L, dtype=i32)

        def group(g, carry):
            off = g * L
            qx = qx_v[pl.ds(off, L)]
            qy = qy_v[pl.ds(off, L)]
            qz = qz_v[pl.ds(off, L)]
            pos0 = (lanes + off) * K
            acc = jnp.zeros((L,), f32)
            facc = jnp.zeros((L,), f32)
            for k in range(K):
                idx = plsc.load_gather(nb_v, [pos0 + k])
                sx = plsc.load_gather(sx_v, [idx])
                sy = plsc.load_gather(sy_v, [idx])
                sz = plsc.load_gather(sz_v, [idx])
                dx = sx - qx
                dy = sy - qy
                dz = sz - qz
                w = jnp.exp((dx * dx + dy * dy + dz * dz) * neg_inv)
                w_v[k, pl.ds(off, L)] = w
                acc = acc + w
                if with_feat:
                    facc = facc + w * plsc.load_gather(f_v, [idx])
            inv = 1.0 / (acc + 1e-8)
            for k in range(K):
                plsc.store_scatter(wr_v, [pos0 + k],
                                   w_v[k, pl.ds(off, L)] * inv)
            if with_feat:
                lv = l_v[pl.ds(0, L)]
                a_v[pl.ds(off, L)] = (lv[0] * facc + lv[1] * acc) * inv
            return carry

        lax.fori_loop(0, G, group, 0)
        pltpu.sync_copy(wr_v, wout.at[pl.ds(base * K, chunk * K)])
        if with_feat:
            pltpu.sync_copy(a_v, aout.at[pl.ds(base, chunk)])

    return pl.kernel(body, out_type=out_type, mesh=_mesh(),
                     scratch_types=scratch,
                     compiler_params=pltpu.CompilerParams(
                         needs_layout_passes=False))


def _sc_weights(spT, qpT, nbf, sigma, feat=None, lift=None):
    ns_pad = spT.shape[1]
    nq_pad = nbf.shape[0] // KNN
    k = _make_weights(ns_pad, nq_pad, sigma, feat is not None)
    args = (spT[0], spT[1], spT[2], qpT[0], qpT[1], qpT[2], nbf)
    if feat is None:
        (wf,) = k(*args)
        return wf.reshape(nq_pad, KNN)
    wf, agg0 = k(*args, feat, lift)
    return wf.reshape(nq_pad, KNN), agg0


def _make_gred(ns_pad, c, nq_pad, op, dtype=f32):
    chunk = nq_pad // NW
    K = KNN
    esz = 2 if dtype == jnp.bfloat16 else 4
    LV = 32 if esz == 2 else L
    nv = c // LV
    CB = min(nv, 16)
    nblk = nv // CB
    bq = max(1, min(4, chunk, 131072 // (K * c * esz)))
    while chunk % bq:
        bq //= 2
    BK = bq * K
    NBAT = chunk // bq

    scratch = [
        pltpu.VMEM((chunk * K,), i32),
        pltpu.VMEM((BK, c), dtype),
        pltpu.VMEM((BK, c), dtype),
        pltpu.VMEM((chunk, c), dtype),
        pltpu.SemaphoreType.DMA,
        pltpu.SemaphoreType.DMA,
    ]
    if op == "wsum":
        scratch.insert(1, pltpu.VMEM((chunk * K,), f32))

    def body(*refs):
        if op == "wsum":
            x_hbm, nbf_hbm, w_hbm, out_hbm, nb_v, w_v, r0, r1, o_v, s0, s1 = refs
        else:
            x_hbm, nbf_hbm, out_hbm, nb_v, r0, r1, o_v, s0, s1 = refs
        base = _wid() * chunk
        pltpu.sync_copy(nbf_hbm.at[pl.ds(base * K, chunk * K)], nb_v)
        if op == "wsum":
            pltpu.sync_copy(w_hbm.at[pl.ds(base * K, chunk * K)], w_v)

        def start(b, buf, sem):
            pltpu.async_copy(x_hbm.at[nb_v.at[pl.ds(b * BK, BK)]], buf, sem)

        def wait(buf, sem):
            pltpu.make_async_copy(x_hbm.at[nb_v.at[pl.ds(0, BK)]], buf,
                                  sem).wait()

        def compute(b, buf):
            def per_q(iq, carry):
                q = b * bq + iq
                r0k = iq * K

                def do_block(cb, carry2):
                    off = cb * (CB * LV)
                    if op == "wsum":
                        acc = [jnp.zeros((L,), f32)] * CB
                        for k in range(K):
                            wk = plsc.load_gather(
                                w_v, [jnp.full((L,), q * K + k, i32)])
                            for j in range(CB):
                                acc[j] = acc[j] + (
                                    buf[r0k + k, pl.ds(off + j * L, L)] * wk)
                    else:
                        acc = [buf[r0k, pl.ds(off + j * LV, LV)]
                               for j in range(CB)]
                        for k in range(1, K):
                            for j in range(CB):
                                acc[j] = jnp.maximum(
                                    acc[j],
                                    buf[r0k + k, pl.ds(off + j * LV, LV)])
                    for j in range(CB):
                        o_v[q, pl.ds(off + j * LV, LV)] = acc[j]
                    return carry2

                if nblk == 1:
                    do_block(0, 0)
                else:
                    lax.fori_loop(0, nblk, do_block, 0)
                return carry

            if bq == 1:
                per_q(0, 0)
            else:
                lax.fori_loop(0, bq, per_q, 0)

        G2 = NBAT // 2
        start(0, r0, s0)

        def pair(g, carry):
            b0 = 2 * g
            start(b0 + 1, r1, s1)
            wait(r0, s0)
            compute(b0, r0)
            start(b0 + 2, r0, s0)
            wait(r1, s1)
            compute(b0 + 1, r1)
            return carry

        lax.fori_loop(0, G2 - 1, pair, 0)
        start(NBAT - 1, r1, s1)
        wait(r0, s0)
        compute(NBAT - 2, r0)
        wait(r1, s1)
        compute(NBAT - 1, r1)

        pltpu.sync_copy(o_v, out_hbm.at[pl.ds(base, chunk)])

    return pl.kernel(
        body, out_type=jax.ShapeDtypeStruct((nq_pad, c), dtype),
        mesh=_mesh(), scratch_types=scratch,
        compiler_params=pltpu.CompilerParams(
            needs_layout_passes=False, use_tc_tiling_on_sc=False))


def _sc_wagg(x, nb, w):
    k = _make_gred(x.shape[0], x.shape[1], nb.shape[0], "wsum")
    return k(x, nb.reshape(-1), w.reshape(-1))


def _sc_maxgather(x, nb):
    k = _make_gred(x.shape[0], x.shape[1], nb.shape[0], "max", x.dtype)
    return k(x, nb.reshape(-1))


def _make_rowgather(ns_pad, c, nq_pad, dtype=f32):
    chunk = nq_pad // NW

    def body(x_hbm, up_hbm, out_hbm, idx_v, rows_v, sem):
        base = _wid() * chunk
        pltpu.sync_copy(up_hbm.at[pl.ds(base, chunk)], idx_v)
        pltpu.async_copy(x_hbm.at[idx_v], rows_v, sem).wait()
        pltpu.sync_copy(rows_v, out_hbm.at[pl.ds(base, chunk)])

    return pl.kernel(
        body, out_type=jax.ShapeDtypeStruct((nq_pad, c), dtype),
        mesh=_mesh(),
        scratch_types=[
            pltpu.VMEM((chunk,), i32),
            pltpu.VMEM((chunk, c), dtype),
            pltpu.SemaphoreType.DMA,
        ],
        compiler_params=pltpu.CompilerParams(
            needs_layout_passes=False, use_tc_tiling_on_sc=False))


def _sc_rowgather(x, up):
    return _make_rowgather(x.shape[0], x.shape[1], up.shape[0], x.dtype)(x, up)


_PREC = lax.Precision.DEFAULT


def _dot(a, b):
    return jnp.dot(a, b, preferred_element_type=f32, precision=_PREC)


def _bn(n):
    return 256 if n % 256 == 0 else n


def _mm(x, W, b, relu=False, x2=None, W2=None, out_dtype=f32):
    n, ci = x.shape
    co = W.shape[1]
    bn = _bn(n)
    grid = (n // bn,)
    b2 = b.reshape(1, co)
    dual = x2 is not None

    def body(*refs):
        if dual:
            x_ref, w_ref, b_ref, x2_ref, w2_ref, o_ref = refs
        else:
            x_ref, w_ref, b_ref, o_ref = refs
        acc = _dot(x_ref[...].astype(f32), w_ref[...]) + b_ref[...]
        if dual:
            acc = acc + _dot(x2_ref[...].astype(f32), w2_ref[...])
        if relu:
            acc = jnp.maximum(acc, 0.0)
        o_ref[...] = acc.astype(out_dtype)

    in_specs = [
        pl.BlockSpec((bn, ci), lambda i: (i, 0)),
        pl.BlockSpec((ci, co), lambda i: (0, 0)),
        pl.BlockSpec((1, co), lambda i: (0, 0)),
    ]
    args = [x, W, b2]
    if dual:
        ci2 = x2.shape[1]
        in_specs += [
            pl.BlockSpec((bn, ci2), lambda i: (i, 0)),
            pl.BlockSpec((ci2, co), lambda i: (0, 0)),
        ]
        args += [x2, W2]

    return pl.pallas_call(
        body, grid=grid,
        in_specs=in_specs,
        out_specs=pl.BlockSpec((bn, co), lambda i: (i, 0)),
        out_shape=jax.ShapeDtypeStruct((n, co), out_dtype),
        compiler_params=pltpu.CompilerParams(
            dimension_semantics=("parallel",)),
    )(*args)


def _mmtail(ha, Wc, bc, Wu, bu, sc, Wsc):
    n, m = ha.shape
    o = Wu.shape[1]
    ci2 = sc.shape[1]
    bn = _bn(n)
    grid = (n // bn,)

    def body(ha_ref, wc_ref, bc_ref, wu_ref, bu_ref, sc_ref, wsc_ref, o_ref):
        h2 = jnp.maximum(_dot(ha_ref[...], wc_ref[...]) + bc_ref[...], 0.0)
        acc = _dot(h2, wu_ref[...]) + bu_ref[...]
        acc = acc + _dot(sc_ref[...].astype(f32), wsc_ref[...])
        o_ref[...] = jnp.maximum(acc, 0.0)

    return pl.pallas_call(
        body, grid=grid,
        in_specs=[
            pl.BlockSpec((bn, m), lambda i: (i, 0)),
            pl.BlockSpec((m, m), lambda i: (0, 0)),
            pl.BlockSpec((1, m), lambda i: (0, 0)),
            pl.BlockSpec((m, o), lambda i: (0, 0)),
            pl.BlockSpec((1, o), lambda i: (0, 0)),
            pl.BlockSpec((bn, ci2), lambda i: (i, 0)),
            pl.BlockSpec((ci2, o), lambda i: (0, 0)),
        ],
        out_specs=pl.BlockSpec((bn, o), lambda i: (i, 0)),
        out_shape=jax.ShapeDtypeStruct((n, o), f32),
        compiler_params=pltpu.CompilerParams(
            dimension_semantics=("parallel",)),
    )(ha, Wc, bc.reshape(1, m), Wu, bu.reshape(1, o), sc, Wsc)


def _pad_rows(a, P):
    return jnp.pad(a, ((0, P - a.shape[0]),) + ((0, 0),) * (a.ndim - 1))


def _pad_idx(a, P, ns):
    a = a.astype(i32)
    npad = P - a.shape[0]
    if npad == 0:
        return a
    K2 = a.shape[1]
    pad = (jnp.arange(npad * K2, dtype=i32) * 7919) % ns
    return jnp.concatenate([a, pad.reshape(npad, K2)], axis=0)


def _resnet(x, nb, w, p, strided):
    h = _mm(x, p["Wd"], p["bd"], relu=True)
    ha = _sc_wagg(h, nb, w)
    sc = _sc_maxgather(x.astype(jnp.bfloat16), nb) if strided else x
    return _mmtail(ha, p["Wc"], p["bc"], p["Wu"], p["bu"], sc, p["Wsc"])


def kernel(feats, points_0, points_1, points_2, points_3,
           neighbors_0, neighbors_1, neighbors_2, neighbors_3,
           subsampling_0, subsampling_1, subsampling_2,
           upsampling_0, upsampling_1, upsampling_2, params):
    pts = [points_0, points_1, points_2, points_3]
    nbs = [neighbors_0, neighbors_1, neighbors_2, neighbors_3]
    subs = [subsampling_0, subsampling_1, subsampling_2]
    n = [p.shape[0] for p in pts]
    Pw = [-(-m // 512) * 512 for m in n]
    Pg = [-(-m // 256) * 256 for m in n]

    ptT = [_pad_rows(p, Pw[i]).T for i, p in enumerate(pts)]
    nbw = [_pad_idx(a, Pw[i], n[i]) for i, a in enumerate(nbs)]
    sbw = [_pad_idx(a, Pw[i + 1], n[i]) for i, a in enumerate(subs)]
    nbg = [a[:Pg[i]] for i, a in enumerate(nbw)]
    sbg = [a[:Pg[i + 1]] for i, a in enumerate(sbw)]
    u1 = _pad_idx(upsampling_1, Pg[1], n[2])[:, 0]
    u2 = _pad_idx(upsampling_2, Pg[2], n[3])[:, 0]

    pr = params
    featp = _pad_rows(feats, Pw[0])[:, 0]
    lift = jnp.zeros((L,), f32).at[0].set(pr["lift_W"][0, 0]).at[1].set(
        pr["lift_b"][0])

    s = 2.0
    w0, agg0 = _sc_weights(ptT[0], ptT[0], nbw[0].reshape(-1), s, featp, lift)
    a8 = jnp.pad(agg0[:Pg[0], None], ((0, 0), (0, 7)))
    W8 = jnp.pad(pr["s1_W"], ((0, 7), (0, 0)))
    x1 = _mm(a8, W8, pr["s1_b"], relu=True)
    x1 = _resnet(x1, nbg[0], w0[:Pg[0]], pr["e1_2"], False)

    ws0 = _sc_weights(ptT[0], ptT[1], sbw[0].reshape(-1), s)
    x2 = _resnet(x1, sbg[0], ws0[:Pg[1]], pr["e2_1"], True)
    w1 = _sc_weights(ptT[1], ptT[1], nbw[1].reshape(-1), 2 * s)[:Pg[1]]
    x2 = _resnet(x2, nbg[1], w1, pr["e2_2"], False)
    x2 = _resnet(x2, nbg[1], w1, pr["e2_3"], False)
    h = _mm(x2, pr["inv2_W"], pr["inv2_b"], relu=True,
            out_dtype=jnp.bfloat16)
    x2_inv = _sc_maxgather(h, nbg[1])

    ws1 = _sc_weights(ptT[1], ptT[2], sbw[1].reshape(-1), 2 * s)
    x3 = _resnet(x2, sbg[1], ws1[:Pg[2]], pr["e3_1"], True)
    w2 = _sc_weights(ptT[2], ptT[2], nbw[2].reshape(-1), 4 * s)[:Pg[2]]
    x3 = _resnet(x3, nbg[2], w2, pr["e3_2"], False)
    x3 = _resnet(x3, nbg[2], w2, pr["e3_3"], False)
    h = _mm(x3, pr["inv3_W"], pr["inv3_b"], relu=True,
            out_dtype=jnp.bfloat16)
    x3_inv = _sc_maxgather(h, nbg[2])

    ws2 = _sc_weights(ptT[2], ptT[3], sbw[2].reshape(-1), 4 * s)
    x4 = _resnet(x3, sbg[2], ws2[:Pg[3]], pr["e4_1"], True)
    w3 = _sc_weights(ptT[3], ptT[3], nbw[3].reshape(-1), 8 * s)[:Pg[3]]
    x4 = _resnet(x4, nbg[3], w3, pr["e4_2"], False)
    x4 = _resnet(x4, nbg[3], w3, pr["e4_3"], False)
    h = _mm(x4, pr["inv4_W"], pr["inv4_b"], relu=True,
            out_dtype=jnp.bfloat16)
    l4b = _sc_maxgather(h, nbg[3])
    l4 = l4b.astype(f32)

    d = l4.shape[1]
    l3g = _sc_rowgather(l4b, u2)
    l3 = _mm(l3g, pr["dec3_W"][:d], pr["dec3_b"], relu=True,
             x2=x3_inv, W2=pr["dec3_W"][d:])
    l2g = _sc_rowgather(l3, u1)
    d3 = l3.shape[1]
    l2 = _mm(l2g, pr["dec2_W"][:d3], pr["dec2_b"], relu=False,
             x2=x2_inv, W2=pr["dec2_W"][d3:])

    return (l2[:n[1]], l3[:n[2]], l4[:n[3]])
